# Initial kernel scaffold; baseline (speedup 1.0000x reference)
import jax
import jax.numpy as jnp
from jax.experimental import pallas as pl


def kernel(x, edge_index, batch, W1a, b1a, g1, be1, m1, v1, W1b, b1b, W2a, b2a, g2, be2, m2, v2, W2b, b2b, Wl, bl):
    raise NotImplementedError("write your pallas kernel here")


# speedup vs baseline: 2.4336x; 2.4336x over previous
import functools

import jax
import jax.numpy as jnp
from jax import lax
from jax.experimental import pallas as pl
from jax.experimental.pallas import tpu as pltpu
from jax.experimental.pallas import tpu_sc as plsc

HID = 64
NG = 128
N_PAD = 10240
BM = 1024
CHUNK = 128
NSUB = 16
NCORE = 2

_F32 = jnp.float32
_HI = lax.Precision.HIGHEST
_DEF = lax.Precision.DEFAULT


def _sc_segment_sum(table_r, idx, dst, feat, fch):
    nchf = feat // fch
    chunks_per_core = nchf // NCORE
    e_pad = dst.shape[0]
    edges_per_w = e_pad // NSUB
    nchunk = edges_per_w // CHUNK
    rows_per_sub = N_PAD // NSUB
    mesh = plsc.VectorSubcoreMesh(core_axis_name="c", subcore_axis_name="s")

    @functools.partial(
        pl.kernel,
        out_type=jax.ShapeDtypeStruct((N_PAD, feat), _F32),
        mesh=mesh,
        scratch_types=[
            pltpu.VMEM((CHUNK,), jnp.int32),
            pltpu.VMEM((CHUNK,), jnp.int32),
            pltpu.VMEM((CHUNK, fch), _F32),
            pltpu.VMEM((CHUNK, fch), _F32),
            pltpu.VMEM_SHARED((N_PAD, fch), _F32),
        ],
        compiler_params=pltpu.CompilerParams(use_tc_tiling_on_sc=False),
    )
    def seg_kernel(table_hbm, idx_hbm, dst_hbm, out_hbm,
                   src_v, dst_v, rows_v, zero_v, acc_sh):
        c = lax.axis_index("c")
        s = lax.axis_index("s")
        row0 = s * rows_per_sub

        @pl.loop(0, CHUNK)
        def _(r):
            @pl.loop(0, fch, step=16)
            def _(col):
                zero_v[r, pl.ds(col, 16)] = jnp.zeros((16,), _F32)

        for k in range(chunks_per_core):
            cc = c * chunks_per_core + k
            @pl.loop(0, rows_per_sub // CHUNK)
            def _(z):
                pltpu.sync_copy(zero_v,
                                acc_sh.at[pl.ds(row0 + z * CHUNK, CHUNK)])
            plsc.subcore_barrier()

            base = s * edges_per_w

            @pl.loop(0, nchunk)
            def _(j):
                off = base + j * CHUNK
                pltpu.sync_copy(idx_hbm.at[cc].at[pl.ds(off, CHUNK)], src_v)
                pltpu.sync_copy(dst_hbm.at[pl.ds(off, CHUNK)], dst_v)
                pltpu.sync_copy(table_hbm.at[src_v], rows_v)
                pltpu.sync_copy(rows_v, acc_sh.at[dst_v], add=True)

            plsc.subcore_barrier()
            pltpu.sync_copy(acc_sh.at[pl.ds(row0, rows_per_sub)],
                            out_hbm.at[pl.ds(row0, rows_per_sub),
                                       pl.ds(cc * fch, fch)])
            plsc.subcore_barrier()

    return seg_kernel(table_r, idx, dst)


def _tc_gin_layer(x, agg, P, Wa, Wb, n_real, out_width):
    K = Wa.shape[0]

    def body(x_ref, a_ref, P_ref, wa_ref, wb_ref, o_ref):
        i = pl.program_id(0)
        Pm = P_ref[...]
        h = jnp.dot(x_ref[...] + a_ref[...], wa_ref[...],
                    preferred_element_type=_F32, precision=_DEF) + Pm[0]
        h = (h - Pm[3]) / jnp.sqrt(Pm[4] + 1e-5) * Pm[1] + Pm[2]
        h = jnp.maximum(h, 0.0)
        o = jnp.dot(h, wb_ref[...], preferred_element_type=_F32,
                    precision=_DEF) + Pm[5]
        o = jnp.maximum(o, 0.0)
        rows = i * BM + lax.broadcasted_iota(jnp.int32, (BM, 1), 0)
        o_ref[...] = jnp.where(rows < n_real, o, 0.0)

    return pl.pallas_call(
        body,
        grid=(N_PAD // BM,),
        in_specs=[pl.BlockSpec((BM, K), lambda i: (i, 0)),
                  pl.BlockSpec((BM, K), lambda i: (i, 0)),
                  pl.BlockSpec((8, HID), lambda i: (0, 0)),
                  pl.BlockSpec((K, HID), lambda i: (0, 0)),
                  pl.BlockSpec((HID, out_width), lambda i: (0, 0))],
        out_specs=pl.BlockSpec((BM, out_width), lambda i: (i, 0)),
        out_shape=jax.ShapeDtypeStruct((N_PAD, out_width), _F32),
        compiler_params=pltpu.CompilerParams(
            dimension_semantics=("parallel",)),
    )(x, agg, P, Wa, Wb)


def _tc_layer2_pool(x1, agg2, P, Wa, Wb, batch3d, Wl, bl2):
    grid = N_PAD // BM

    def body(x1_ref, a_ref, P_ref, wa_ref, wb_ref, b_ref, wl_ref, bl_ref,
             out_ref, pool_acc):
        i = pl.program_id(0)
        Pm = P_ref[...]
        h = jnp.dot(x1_ref[...] + a_ref[...], wa_ref[...],
                    preferred_element_type=_F32, precision=_DEF) + Pm[0]
        h = (h - Pm[3]) / jnp.sqrt(Pm[4] + 1e-5) * Pm[1] + Pm[2]
        h = jnp.maximum(h, 0.0)
        x2 = jnp.dot(h, wb_ref[...], preferred_element_type=_F32,
                     precision=_DEF) + Pm[5]
        x2 = jnp.maximum(x2, 0.0)
        xjk = jnp.concatenate([x1_ref[...], x2], axis=1)
        bids = b_ref[0, 0, :]
        oh = (bids[:, None] ==
              lax.broadcasted_iota(jnp.int32, (BM, NG), 1)).astype(_F32)
        contrib = lax.dot_general(oh, xjk, (((0,), (0,)), ((), ())),
                                  precision=_HI,
                                  preferred_element_type=_F32)

        @pl.when(i == 0)
        def _():
            pool_acc[...] = jnp.zeros((NG, 2 * HID), _F32)

        pool_acc[...] += contrib

        @pl.when(i == grid - 1)
        def _():
            out_ref[...] = jnp.dot(pool_acc[...], wl_ref[...],
                                   preferred_element_type=_F32,
                                   precision=_DEF) + bl_ref[...]

    blk = pl.BlockSpec((BM, HID), lambda i: (i, 0))
    return pl.pallas_call(
        body,
        grid=(grid,),
        in_specs=[blk, blk,
                  pl.BlockSpec((8, HID), lambda i: (0, 0)),
                  pl.BlockSpec((HID, HID), lambda i: (0, 0)),
                  pl.BlockSpec((HID, HID), lambda i: (0, 0)),
                  pl.BlockSpec((1, 1, BM), lambda i: (i, 0, 0)),
                  pl.BlockSpec((2 * HID, 2), lambda i: (0, 0)),
                  pl.BlockSpec((1, 2), lambda i: (0, 0))],
        out_specs=pl.BlockSpec((NG, 2), lambda i: (0, 0)),
        out_shape=jax.ShapeDtypeStruct((NG, 2), _F32),
        scratch_shapes=[pltpu.VMEM((NG, 2 * HID), _F32)],
    )(x1, agg2, P, Wa, Wb, batch3d, Wl, bl2)


def _chunk_indices(src_pad, nchf):
    return (src_pad[None, :] * nchf
            + jnp.arange(nchf, dtype=jnp.int32)[:, None])


def kernel(x, edge_index, batch,
           W1a, b1a, g1, be1, m1, v1, W1b, b1b,
           W2a, b2a, g2, be2, m2, v2, W2b, b2b,
           Wl, bl):
    n = x.shape[0]
    feat = x.shape[1]
    e = edge_index.shape[1]
    src = edge_index[0].astype(jnp.int32)
    dst = edge_index[1].astype(jnp.int32)
    unit = CHUNK * NSUB
    e_pad = ((e + unit - 1) // unit) * unit
    src_pad = jnp.concatenate([src, jnp.zeros((e_pad - e,), jnp.int32)])
    dst_pad = jnp.concatenate([dst, jnp.full((e_pad - e,), n, jnp.int32)])
    batch_pad = jnp.concatenate(
        [batch.astype(jnp.int32), jnp.full((N_PAD - n,), NG, jnp.int32)]
    ).reshape(N_PAD // BM, 1, BM)
    zpad = jnp.zeros((2, HID), _F32)
    P1 = jnp.concatenate([jnp.stack([b1a, g1, be1, m1, v1, b1b]), zpad])
    P2 = jnp.concatenate([jnp.stack([b2a, g2, be2, m2, v2, b2b]), zpad])

    fch1 = feat // 6
    x_r = x.reshape(n * 6, fch1)
    agg1 = _sc_segment_sum(x_r, _chunk_indices(src_pad, 6), dst_pad,
                           feat, fch1)
    x1 = _tc_gin_layer(x, agg1, P1, W1a, W1b, n, HID)

    fch2 = HID // 2
    x1_r = x1.reshape(N_PAD * 2, fch2)
    agg2 = _sc_segment_sum(x1_r, _chunk_indices(src_pad, 2), dst_pad,
                           HID, fch2)
    return _tc_layer2_pool(x1, agg2, P2, W2a, W2b, batch_pad,
                           Wl, bl.reshape(1, 2))
